# baseline (speedup 1.0000x reference)
import functools

import jax
import jax.numpy as jnp
from jax.experimental import pallas as pl
from jax.experimental.pallas import tpu as pltpu

N = 10000
L = 10
BLK = 512
NPAD = 10240


def _mlp_body(res_ref, mask_ref, w1_ref, b1_ref, w2_ref, b2_ref, w3_ref, b3_ref,
              out_ref, sum_ref):
    i = pl.program_id(0)
    res = res_ref[...]
    h1 = jnp.maximum(jnp.dot(res, w1_ref[...].T, preferred_element_type=jnp.float32)
                     + b1_ref[...], 0.0)
    h2 = jnp.maximum(jnp.dot(h1, w2_ref[...].T, preferred_element_type=jnp.float32)
                     + b2_ref[...], 0.0)
    r = jnp.sum(h2 * w3_ref[...], axis=1, keepdims=True) + b3_ref[0]
    row = jax.lax.broadcasted_iota(jnp.int32, (BLK, 1), 0) + i * BLK
    valid = row < N
    r = jnp.where(valid, r, 0.0)
    out_ref[...] = r * mask_ref[...]

    @pl.when(i == 0)
    def _():
        sum_ref[...] = jnp.zeros_like(sum_ref)

    sum_ref[...] += jnp.sum(r).reshape(1, 1)


def _mlp_head(res, mask, W1, b1, W2, b2, W3, b3):
    res_p = jnp.zeros((NPAD, 3), jnp.float32).at[:N].set(res)
    mask_p = jnp.zeros((NPAD, 1), jnp.float32).at[:N, 0].set(mask)
    W1e = W1[:, :3]
    grid = NPAD // BLK
    out, rsum = pl.pallas_call(
        _mlp_body,
        grid=(grid,),
        in_specs=[
            pl.BlockSpec((BLK, 3), lambda i: (i, i * 0)),
            pl.BlockSpec((BLK, 1), lambda i: (i, i * 0)),
            pl.BlockSpec((16, 3), lambda i: (i * 0, i * 0)),
            pl.BlockSpec((1, 16), lambda i: (i * 0, i * 0)),
            pl.BlockSpec((32, 16), lambda i: (i * 0, i * 0)),
            pl.BlockSpec((1, 32), lambda i: (i * 0, i * 0)),
            pl.BlockSpec((1, 32), lambda i: (i * 0, i * 0)),
            pl.BlockSpec((1,), lambda i: (i * 0,), memory_space=pltpu.MemorySpace.SMEM),
        ],
        out_specs=[
            pl.BlockSpec((BLK, 1), lambda i: (i, i * 0)),
            pl.BlockSpec((1, 1), lambda i: (i * 0, i * 0)),
        ],
        out_shape=[
            jax.ShapeDtypeStruct((NPAD, 1), jnp.float32),
            jax.ShapeDtypeStruct((1, 1), jnp.float32),
        ],
    )(res_p, mask_p, W1e, b1.reshape(1, 16), W2, b2.reshape(1, 32),
      W3.reshape(1, 32), b3)
    return out[:N, 0], rsum[0, 0]


def kernel(x, edge_index, mask, gat_W, gat_as, gat_ad, gat_b, Wqkv, bqkv, Wo, bo,
           W1, b1, W2, b2, W3, b3, Wc, bc):
    src = jnp.concatenate([edge_index[0], jnp.arange(N, dtype=edge_index.dtype)])
    dst = jnp.concatenate([edge_index[1], jnp.arange(N, dtype=edge_index.dtype)])
    h = x
    for l in range(L):
        hp = h @ gat_W[l].T
        a_s = (hp * gat_as[l]).sum(-1)
        a_d = (hp * gat_ad[l]).sum(-1)
        e = a_s[src] + a_d[dst]
        e = jnp.where(e > 0, e, 0.2 * e)
        m = jax.ops.segment_max(e, dst, num_segments=N)
        m = jnp.where(jnp.isfinite(m), m, 0.0)
        ex = jnp.exp(e - m[dst])
        s = jax.ops.segment_sum(ex, dst, num_segments=N)
        alpha = ex / (s[dst] + 1e-16)
        out = jax.ops.segment_sum(alpha[:, None] * hp[src], dst, num_segments=N) + gat_b[l]
        h = jax.nn.relu(out) if l < L - 1 else out
    q = h @ Wqkv[0:3].T + bqkv[0:3]
    k = h @ Wqkv[3:6].T + bqkv[3:6]
    v = h @ Wqkv[6:9].T + bqkv[6:9]
    qh = q.T[:, :, None]
    kh = k.T[:, :, None]
    vh = v.T[:, :, None]
    scores = jnp.matmul(qh, kh.transpose(0, 2, 1))
    attn = jax.nn.softmax(scores, axis=-1)
    ctx = jnp.matmul(attn, vh)[:, :, 0].T
    res = ctx @ Wo.T + bo
    new_results, rsum = _mlp_head(res.astype(jnp.float32), mask, W1, b1, W2, b2, W3, b3)
    value = Wc[0, 0] * (rsum / N) + bc[0]
    return new_results, value
